# baseline (speedup 1.0000x reference)
import functools

import jax
import jax.numpy as jnp
from jax import lax
from jax.experimental import pallas as pl
from jax.experimental.pallas import tpu as pltpu
from jax.experimental.pallas import tpu_sc as plsc

N_NODES = 50000
D_FEAT = 256
N_GRAPHS = 512

NUM_CORES = 2
NUM_SUBCORES = 16
NUM_WORKERS = NUM_CORES * NUM_SUBCORES
SEG_PER_W = N_GRAPHS // NUM_WORKERS
LANES = 16
NBLK = D_FEAT // LANES
CHUNK = 96

_mesh = plsc.VectorSubcoreMesh(core_axis_name="c", subcore_axis_name="s")


@functools.partial(
    pl.kernel,
    out_type=jax.ShapeDtypeStruct((N_GRAPHS, D_FEAT), jnp.float32),
    mesh=_mesh,
    scratch_types=[
        pltpu.VMEM((N_NODES + LANES,), jnp.int32),
        pltpu.VMEM((CHUNK, D_FEAT), jnp.float32),
        pltpu.VMEM((CHUNK, D_FEAT), jnp.float32),
        pltpu.VMEM(((SEG_PER_W + 1) * D_FEAT,), jnp.float32),
        pltpu.VMEM((D_FEAT,), jnp.float32),
        pltpu.SemaphoreType.DMA,
        pltpu.SemaphoreType.DMA,
    ],
)
def _pool(
    feat_hbm, nid_hbm, out_hbm, nid_v, rows0, rows1, acc_v, regb, sem0, sem1
):
    wid = lax.axis_index("s") * NUM_CORES + lax.axis_index("c")
    seg_lo = pl.multiple_of(wid * SEG_PER_W, SEG_PER_W)

    nid_cp = pltpu.async_copy(nid_hbm, nid_v.at[pl.ds(0, N_NODES)], sem0)
    nid_v[pl.ds(N_NODES, LANES)] = jnp.full((LANES,), N_GRAPHS, jnp.int32)

    zeros = jnp.zeros((LANES,), jnp.float32)

    def zero_body(s, carry):
        acc_v[pl.ds(s * LANES, LANES)] = zeros
        return carry

    lax.fori_loop(0, (SEG_PER_W + 1) * NBLK, zero_body, 0)
    for f in range(NBLK):
        regb[pl.ds(f * LANES, LANES)] = zeros
    nid_cp.wait()

    def bsearch(target):
        def search_body(_, carry):
            lo_c, hi_c = carry
            mid = (lo_c + hi_c) >> 1
            val = nid_v[pl.ds(mid, LANES)][0]
            less = val < target
            return (
                jnp.where(less, mid + 1, lo_c),
                jnp.where(less, hi_c, mid),
            )

        lo_f, _ = lax.fori_loop(0, 17, search_body, (0, N_NODES))
        return lo_f

    row_lo = bsearch(seg_lo)
    row_hi = bsearch(seg_lo + SEG_PER_W)

    start = (row_lo >> 3) << 3
    npairs = (row_hi - start + 2 * CHUNK - 1) // (2 * CHUNK)

    def chunk_off(k):
        u = start + k * CHUNK
        return u, pl.multiple_of(jnp.minimum(u, N_NODES - CHUNK), 8)

    def dma_start(k, buf, sem):
        _, o = chunk_off(k)
        pltpu.async_copy(feat_hbm.at[pl.ds(o, CHUNK)], buf, sem)

    def dma_wait(buf, sem):
        pltpu.make_async_copy(feat_hbm.at[pl.ds(0, CHUNK)], buf, sem).wait()

    def flush(cur_s):
        for f in range(NBLK):
            sl = pl.ds(f * LANES, LANES)
            plsc.addupdate(
                acc_v.at[pl.ds(cur_s * D_FEAT + f * LANES, LANES)], regb[sl]
            )
            regb[sl] = zeros

    def compute(k, buf, cur):
        u, o = chunk_off(k)
        lo_r = jnp.maximum(u, row_lo)

        def grp_body(j, cur_c):
            base = j * LANES
            iv = nid_v[pl.ds(o + base, LANES)]
            first = iv[0]
            uniform = (
                (first == iv[LANES - 1])
                & (o + base >= lo_r)
                & (o + base + LANES <= row_hi)
            )

            def fast(cur_f):
                lseg0 = first - seg_lo

                @pl.when(lseg0 != cur_f)
                def _():
                    flush(cur_f)

                for f in range(NBLK):
                    sl = pl.ds(f * LANES, LANES)
                    v = regb[sl]
                    for t in range(LANES):
                        v = v + buf[base + t, sl]
                    regb[sl] = v
                return lseg0

            def slow(cur_f):
                lseg_v = iv - seg_lo
                for t in range(LANES):
                    r = o + base + t
                    valid = (r >= lo_r) & (r < row_hi)
                    nxt = jnp.where(valid, lseg_v[t], jnp.int32(SEG_PER_W))

                    @pl.when(nxt != cur_f)
                    def _(cur_s=cur_f):
                        flush(cur_s)

                    for f in range(NBLK):
                        sl = pl.ds(f * LANES, LANES)
                        plsc.addupdate(regb.at[sl], buf[base + t, sl])
                    cur_f = nxt
                return cur_f

            return lax.cond(uniform, fast, slow, cur_c)

        return lax.fori_loop(0, CHUNK // LANES, grp_body, cur)

    @pl.when(npairs > 0)
    def _():
        dma_start(0, rows0, sem0)

    def pair_body(p, cur_c):
        k0 = 2 * p
        dma_start(k0 + 1, rows1, sem1)
        dma_wait(rows0, sem0)
        cur_c = compute(k0, rows0, cur_c)

        @pl.when(p + 1 < npairs)
        def _():
            dma_start(k0 + 2, rows0, sem0)

        dma_wait(rows1, sem1)
        cur_c = compute(k0 + 1, rows1, cur_c)
        return cur_c

    cur0 = jnp.int32(SEG_PER_W)
    cur_f = lax.fori_loop(0, npairs, pair_body, cur0)

    flush(cur_f)

    for s in range(SEG_PER_W):
        pltpu.sync_copy(
            acc_v.at[pl.ds(s * D_FEAT, D_FEAT)], out_hbm.at[seg_lo + s]
        )


def kernel(graph, feature, node_id, edge_id):
    return _pool(feature, node_id)


# speedup vs baseline: 2.7485x; 2.7485x over previous
import functools

import jax
import jax.numpy as jnp
from jax import lax
from jax.experimental import pallas as pl
from jax.experimental.pallas import tpu as pltpu
from jax.experimental.pallas import tpu_sc as plsc

N_NODES = 50000
D_FEAT = 256
N_GRAPHS = 512

NUM_CORES = 2
NUM_SUBCORES = 16
NUM_WORKERS = NUM_CORES * NUM_SUBCORES
SEG_PER_W = N_GRAPHS // NUM_WORKERS
LANES = 16
NBLK = D_FEAT // LANES
CHUNK = 64

_mesh = plsc.VectorSubcoreMesh(core_axis_name="c", subcore_axis_name="s")


@functools.partial(
    pl.kernel,
    out_type=jax.ShapeDtypeStruct((N_GRAPHS * D_FEAT,), jnp.float32),
    mesh=_mesh,
    scratch_types=[
        pltpu.VMEM((N_NODES + LANES,), jnp.int32),
        pltpu.VMEM((CHUNK, D_FEAT), jnp.float32),
        pltpu.VMEM((CHUNK, D_FEAT), jnp.float32),
        pltpu.VMEM(((SEG_PER_W + 1) * D_FEAT,), jnp.float32),
        pltpu.SemaphoreType.DMA,
        pltpu.SemaphoreType.DMA,
    ],
)
def _pool(feat_hbm, nid_hbm, out_hbm, nid_v, rows0, rows1, acc_v, sem0, sem1):
    wid = lax.axis_index("s") * NUM_CORES + lax.axis_index("c")
    seg_lo = pl.multiple_of(wid * SEG_PER_W, SEG_PER_W)

    nid_cp = pltpu.async_copy(nid_hbm, nid_v.at[pl.ds(0, N_NODES)], sem0)
    nid_v[pl.ds(N_NODES, LANES)] = jnp.full((LANES,), N_GRAPHS, jnp.int32)

    zeros = jnp.zeros((LANES,), jnp.float32)

    def zero_body(s, carry):
        acc_v[pl.ds(s * LANES, LANES)] = zeros
        return carry

    lax.fori_loop(0, (SEG_PER_W + 1) * NBLK, zero_body, 0)
    nid_cp.wait()

    def bsearch(target):
        def search_body(_, carry):
            lo_c, hi_c = carry
            mid = (lo_c + hi_c) >> 1
            val = nid_v[pl.ds(mid, LANES)][0]
            less = val < target
            return (
                jnp.where(less, mid + 1, lo_c),
                jnp.where(less, hi_c, mid),
            )

        lo_f, _ = lax.fori_loop(0, 17, search_body, (0, N_NODES))
        return lo_f

    row_lo = bsearch(seg_lo)
    row_hi = bsearch(seg_lo + SEG_PER_W)

    start = (row_lo >> 3) << 3
    npairs = (row_hi - start + 2 * CHUNK - 1) // (2 * CHUNK)

    def chunk_off(k):
        u = start + k * CHUNK
        return u, pl.multiple_of(jnp.minimum(u, N_NODES - CHUNK), 8)

    def dma_start(k, buf, sem):
        _, o = chunk_off(k)
        pltpu.async_copy(feat_hbm.at[pl.ds(o, CHUNK)], buf, sem)

    def dma_wait(buf, sem):
        pltpu.make_async_copy(feat_hbm.at[pl.ds(0, CHUNK)], buf, sem).wait()

    def flush(cur_s, regs_s):
        for f in range(NBLK):
            plsc.addupdate(
                acc_v.at[pl.ds(cur_s * D_FEAT + f * LANES, LANES)], regs_s[f]
            )

    def compute(k, buf, cur, regs):
        u, o = chunk_off(k)
        lo_r = jnp.maximum(u, row_lo)

        def grp_body(j, carry):
            cur_c = carry[0]
            regs_c = tuple(carry[1:])
            base = j * LANES
            iv = nid_v[pl.ds(o + base, LANES)]
            lseg_v = iv - seg_lo
            for t in range(LANES):
                r = o + base + t
                valid = (r >= lo_r) & (r < row_hi)
                nxt = jnp.where(valid, lseg_v[t], jnp.int32(SEG_PER_W))
                changed = nxt != cur_c

                @pl.when(changed)
                def _(cur_s=cur_c, regs_s=regs_c):
                    flush(cur_s, regs_s)

                regs_c = tuple(
                    jnp.where(changed, zeros, regs_c[f])
                    + buf[base + t, pl.ds(f * LANES, LANES)]
                    for f in range(NBLK)
                )
                cur_c = nxt
            return (cur_c,) + regs_c

        return lax.fori_loop(0, CHUNK // LANES, grp_body, (cur,) + tuple(regs))

    @pl.when(npairs > 0)
    def _():
        dma_start(0, rows0, sem0)

    def pair_body(p, carry):
        k0 = 2 * p
        dma_start(k0 + 1, rows1, sem1)
        dma_wait(rows0, sem0)
        carry = compute(k0, rows0, carry[0], carry[1:])

        @pl.when(p + 1 < npairs)
        def _():
            dma_start(k0 + 2, rows0, sem0)

        dma_wait(rows1, sem1)
        carry = compute(k0 + 1, rows1, carry[0], carry[1:])
        return carry

    cur0 = jnp.int32(SEG_PER_W)
    init = (cur0,) + tuple(zeros for _ in range(NBLK))
    final = lax.fori_loop(0, npairs, pair_body, init)

    flush(final[0], tuple(final[1:]))

    pltpu.sync_copy(
        acc_v.at[pl.ds(0, SEG_PER_W * D_FEAT)],
        out_hbm.at[pl.ds(seg_lo * D_FEAT, SEG_PER_W * D_FEAT)],
    )


def kernel(graph, feature, node_id, edge_id):
    return _pool(feature, node_id).reshape(N_GRAPHS, D_FEAT)
